# baseline (speedup 1.0000x reference)
import functools

import jax
import jax.numpy as jnp
from jax.experimental import pallas as pl
from jax.experimental.pallas import tpu as pltpu

N = 10000
E = 320000
E_PAD = 327680
N_INTER = 3
NF = 128
NB = 25
CUTOFF = 5.0

B_E = 1024
B_N = 2000

_INTERP = False


def _edge_filter_body(d2_ref, wf1_ref, wf2_ref, bf1_ref, bf2_ref, wc_ref):
    d2 = d2_ref[...]
    r = jnp.sqrt(d2 + 1e-12)
    c = 0.5 * (jnp.cos(r * (jnp.pi / CUTOFF)) + 1.0)
    c = jnp.where(r < CUTOFF, c, 0.0)
    width = CUTOFF / (NB - 1)
    coeff = -0.5 / (width * width)
    k = jax.lax.broadcasted_iota(jnp.int32, (1, NF), 1).astype(jnp.float32)
    off = k * width
    diff = r - off
    fij = jnp.exp(coeff * diff * diff)
    kmask = jax.lax.broadcasted_iota(jnp.int32, (1, NF), 1) < NB
    fij = jnp.where(kmask, fij, 0.0)
    u = jnp.dot(fij, wf1_ref[...], preferred_element_type=jnp.float32) + bf1_ref[...]
    su = u * jax.nn.sigmoid(u)
    w = jnp.dot(su, wf2_ref[...], preferred_element_type=jnp.float32) + bf2_ref[...]
    wc_ref[...] = w * c


def _edge_filter(d2_col, wf1p, wf2, bf1, bf2, e_pad):
    grid = e_pad // B_E
    return pl.pallas_call(
        _edge_filter_body,
        grid=(grid,),
        in_specs=[
            pl.BlockSpec((B_E, 1), lambda i: (i, 0)),
            pl.BlockSpec((NF, NF), lambda i: (0, 0)),
            pl.BlockSpec((NF, NF), lambda i: (0, 0)),
            pl.BlockSpec((1, NF), lambda i: (0, 0)),
            pl.BlockSpec((1, NF), lambda i: (0, 0)),
        ],
        out_specs=pl.BlockSpec((B_E, NF), lambda i: (i, 0)),
        out_shape=jax.ShapeDtypeStruct((e_pad, NF), jnp.float32),
        interpret=_INTERP,
    )(d2_col, wf1p, wf2, bf1, bf2)


def _init_body(z_ref, emb_ref, win0_ref, feat_ref, h_ref):
    z = z_ref[...]
    lane = jax.lax.broadcasted_iota(jnp.int32, (1, NF), 1)
    oh = (z == lane).astype(jnp.float32)
    feat = jnp.dot(oh, emb_ref[...], preferred_element_type=jnp.float32)
    feat_ref[...] = feat
    h_ref[...] = jnp.dot(feat, win0_ref[...], preferred_element_type=jnp.float32)


def _init_feat(z_col, emb_pad, win0):
    grid = N // B_N
    return pl.pallas_call(
        _init_body,
        grid=(grid,),
        in_specs=[
            pl.BlockSpec((B_N, 1), lambda i: (i, 0)),
            pl.BlockSpec((NF, NF), lambda i: (0, 0)),
            pl.BlockSpec((NF, NF), lambda i: (0, 0)),
        ],
        out_specs=[
            pl.BlockSpec((B_N, NF), lambda i: (i, 0)),
            pl.BlockSpec((B_N, NF), lambda i: (i, 0)),
        ],
        out_shape=[
            jax.ShapeDtypeStruct((N, NF), jnp.float32),
            jax.ShapeDtypeStruct((N, NF), jnp.float32),
        ],
        interpret=_INTERP,
    )(z_col, emb_pad, win0)


def _node_body(feat_ref, m_ref, wout1_ref, wout2_ref, bout1_ref, bout2_ref,
               gamma_ref, beta_ref, win_next_ref, feat_out_ref, h_out_ref):
    m = m_ref[...]
    u = jnp.dot(m, wout1_ref[...], preferred_element_type=jnp.float32) + bout1_ref[...]
    su = u * jax.nn.sigmoid(u)
    mm = jnp.dot(su, wout2_ref[...], preferred_element_type=jnp.float32) + bout2_ref[...]
    f = feat_ref[...] + mm
    mu = jnp.mean(f, axis=1, keepdims=True)
    d = f - mu
    var = jnp.mean(d * d, axis=1, keepdims=True)
    fn = gamma_ref[...] * d * jax.lax.rsqrt(var + 1e-5) + beta_ref[...]
    feat_out_ref[...] = fn
    h_out_ref[...] = jnp.dot(fn, win_next_ref[...], preferred_element_type=jnp.float32)


def _node_update(feat, m, wout1, wout2, bout1, bout2, gamma, beta, win_next):
    grid = N // B_N
    return pl.pallas_call(
        _node_body,
        grid=(grid,),
        in_specs=[
            pl.BlockSpec((B_N, NF), lambda i: (i, 0)),
            pl.BlockSpec((B_N, NF), lambda i: (i, 0)),
            pl.BlockSpec((NF, NF), lambda i: (0, 0)),
            pl.BlockSpec((NF, NF), lambda i: (0, 0)),
            pl.BlockSpec((1, NF), lambda i: (0, 0)),
            pl.BlockSpec((1, NF), lambda i: (0, 0)),
            pl.BlockSpec((1, NF), lambda i: (0, 0)),
            pl.BlockSpec((1, NF), lambda i: (0, 0)),
            pl.BlockSpec((NF, NF), lambda i: (0, 0)),
        ],
        out_specs=[
            pl.BlockSpec((B_N, NF), lambda i: (i, 0)),
            pl.BlockSpec((B_N, NF), lambda i: (i, 0)),
        ],
        out_shape=[
            jax.ShapeDtypeStruct((N, NF), jnp.float32),
            jax.ShapeDtypeStruct((N, NF), jnp.float32),
        ],
        interpret=_INTERP,
    )(feat, m, wout1, wout2, bout1, bout2, gamma, beta, win_next)


def kernel(z, pos, edge_index, emb, Wf1, bf1, Wf2, bf2, Win, Wout1, bout1, Wout2, bout2, gamma, beta):
    src = edge_index[0]
    dst = edge_index[1]

    vij = pos[dst] - pos[src]
    d2 = jnp.sum(vij * vij, axis=-1)
    d2_col = jnp.pad(d2, (0, E_PAD - E), constant_values=100.0).reshape(E_PAD, 1)

    wf1p = jnp.pad(Wf1, ((0, 0), (0, NF - NB), (0, 0)))
    emb_pad = jnp.pad(emb, ((0, NF - emb.shape[0]), (0, 0)))
    b2 = lambda b: b.reshape(N_INTER, 1, NF)
    bf1c, bf2c, bout1c, bout2c = b2(bf1), b2(bf2), b2(bout1), b2(bout2)
    gammac, betac = b2(gamma), b2(beta)

    z_col = z.reshape(N, 1).astype(jnp.int32)
    feat, h = _init_feat(z_col, emb_pad, Win[0])

    for i in range(N_INTER):
        wc = _edge_filter(d2_col, wf1p[i], Wf2[i], bf1c[i], bf2c[i], E_PAD)[:E]
        m_e = h[src] * wc
        m = jnp.zeros((N, NF), jnp.float32).at[dst].add(m_e)
        feat, h = _node_update(feat, m, Wout1[i], Wout2[i], bout1c[i], bout2c[i],
                               gammac[i], betac[i], Win[(i + 1) % N_INTER])
    return feat


# speedup vs baseline: 2.0263x; 2.0263x over previous
import functools

import jax
import jax.numpy as jnp
from jax import lax
from jax.experimental import pallas as pl
from jax.experimental.pallas import tpu as pltpu
from jax.experimental.pallas import tpu_sc as plsc

N = 10000
E = 320000
E_PAD = 327680

NC = 2
NS = 16
NW = NC * NS
EPW = E_PAD // NW
ROWS_PER_TILE = 640
N_PADR = 10240
N_INTER = 3
NF = 128
NB = 25
CUTOFF = 5.0

B_E = 1024
B_N = 2000

_INTERP = False


def _edge_filter_body(d2_ref, wf1_ref, wf2_ref, bf1_ref, bf2_ref, wc_ref):
    d2 = d2_ref[...]
    r = jnp.sqrt(d2 + 1e-12)
    c = 0.5 * (jnp.cos(r * (jnp.pi / CUTOFF)) + 1.0)
    c = jnp.where(r < CUTOFF, c, 0.0)
    width = CUTOFF / (NB - 1)
    coeff = -0.5 / (width * width)
    k = jax.lax.broadcasted_iota(jnp.int32, (1, NF), 1).astype(jnp.float32)
    off = k * width
    diff = r - off
    fij = jnp.exp(coeff * diff * diff)
    kmask = jax.lax.broadcasted_iota(jnp.int32, (1, NF), 1) < NB
    fij = jnp.where(kmask, fij, 0.0)
    u = jnp.dot(fij, wf1_ref[...], preferred_element_type=jnp.float32) + bf1_ref[...]
    su = u * jax.nn.sigmoid(u)
    w = jnp.dot(su, wf2_ref[...], preferred_element_type=jnp.float32) + bf2_ref[...]
    wc_ref[...] = w * c


def _edge_filter(d2_col, wf1p, wf2, bf1, bf2, e_pad):
    grid = e_pad // B_E
    return pl.pallas_call(
        _edge_filter_body,
        grid=(grid,),
        in_specs=[
            pl.BlockSpec((B_E, 1), lambda i: (i, 0)),
            pl.BlockSpec((NF, NF), lambda i: (0, 0)),
            pl.BlockSpec((NF, NF), lambda i: (0, 0)),
            pl.BlockSpec((1, NF), lambda i: (0, 0)),
            pl.BlockSpec((1, NF), lambda i: (0, 0)),
        ],
        out_specs=pl.BlockSpec((B_E, NF), lambda i: (i, 0)),
        out_shape=jax.ShapeDtypeStruct((e_pad, NF), jnp.float32),
        interpret=_INTERP,
    )(d2_col, wf1p, wf2, bf1, bf2)


def _init_body(z_ref, emb_ref, win0_ref, feat_ref, h_ref):
    z = z_ref[...]
    lane = jax.lax.broadcasted_iota(jnp.int32, (1, NF), 1)
    oh = (z == lane).astype(jnp.float32)
    feat = jnp.dot(oh, emb_ref[...], preferred_element_type=jnp.float32)
    feat_ref[...] = feat
    h_ref[...] = jnp.dot(feat, win0_ref[...], preferred_element_type=jnp.float32)


def _init_feat(z_col, emb_pad, win0):
    grid = N // B_N
    return pl.pallas_call(
        _init_body,
        grid=(grid,),
        in_specs=[
            pl.BlockSpec((B_N, 1), lambda i: (i, 0)),
            pl.BlockSpec((NF, NF), lambda i: (0, 0)),
            pl.BlockSpec((NF, NF), lambda i: (0, 0)),
        ],
        out_specs=[
            pl.BlockSpec((B_N, NF), lambda i: (i, 0)),
            pl.BlockSpec((B_N, NF), lambda i: (i, 0)),
        ],
        out_shape=[
            jax.ShapeDtypeStruct((N, NF), jnp.float32),
            jax.ShapeDtypeStruct((N, NF), jnp.float32),
        ],
        interpret=_INTERP,
    )(z_col, emb_pad, win0)


def _node_body(feat_ref, m0_ref, m1_ref, wout1_ref, wout2_ref, bout1_ref, bout2_ref,
               gamma_ref, beta_ref, win_next_ref, feat_out_ref, h_out_ref):
    m = m0_ref[...] + m1_ref[...]
    u = jnp.dot(m, wout1_ref[...], preferred_element_type=jnp.float32) + bout1_ref[...]
    su = u * jax.nn.sigmoid(u)
    mm = jnp.dot(su, wout2_ref[...], preferred_element_type=jnp.float32) + bout2_ref[...]
    f = feat_ref[...] + mm
    mu = jnp.mean(f, axis=1, keepdims=True)
    d = f - mu
    var = jnp.mean(d * d, axis=1, keepdims=True)
    fn = gamma_ref[...] * d * jax.lax.rsqrt(var + 1e-5) + beta_ref[...]
    feat_out_ref[...] = fn
    h_out_ref[...] = jnp.dot(fn, win_next_ref[...], preferred_element_type=jnp.float32)


def _node_update(feat, m0, m1, wout1, wout2, bout1, bout2, gamma, beta, win_next):
    grid = N // B_N
    return pl.pallas_call(
        _node_body,
        grid=(grid,),
        in_specs=[
            pl.BlockSpec((B_N, NF), lambda i: (i, 0)),
            pl.BlockSpec((B_N, NF), lambda i: (i, 0)),
            pl.BlockSpec((B_N, NF), lambda i: (i, 0)),
            pl.BlockSpec((NF, NF), lambda i: (0, 0)),
            pl.BlockSpec((NF, NF), lambda i: (0, 0)),
            pl.BlockSpec((1, NF), lambda i: (0, 0)),
            pl.BlockSpec((1, NF), lambda i: (0, 0)),
            pl.BlockSpec((1, NF), lambda i: (0, 0)),
            pl.BlockSpec((1, NF), lambda i: (0, 0)),
            pl.BlockSpec((NF, NF), lambda i: (0, 0)),
        ],
        out_specs=[
            pl.BlockSpec((B_N, NF), lambda i: (i, 0)),
            pl.BlockSpec((B_N, NF), lambda i: (i, 0)),
        ],
        out_shape=[
            jax.ShapeDtypeStruct((N, NF), jnp.float32),
            jax.ShapeDtypeStruct((N, NF), jnp.float32),
        ],
        interpret=_INTERP,
    )(feat, m0, m1, wout1, wout2, bout1, bout2, gamma, beta, win_next)


CE_D2 = 512
CE = 128
ZROWS = 128


def _sc_mesh():
    return plsc.VectorSubcoreMesh(core_axis_name="c", subcore_axis_name="s",
                                  num_cores=NC, num_subcores=NS)


def _sc_d2_body(px_hbm, py_hbm, pz_hbm, src_hbm, dst_hbm, d2_hbm,
                px_v, py_v, pz_v, src_v, dst_v, d2_v):
    cid = lax.axis_index("c")
    sid = lax.axis_index("s")
    wid = sid * NC + cid
    pltpu.sync_copy(px_hbm, px_v)
    pltpu.sync_copy(py_hbm, py_v)
    pltpu.sync_copy(pz_hbm, pz_v)
    lanes = jnp.arange(16, dtype=jnp.int32)
    wbase = wid * EPW

    @pl.loop(0, EPW // CE_D2)
    def _chunk(ci):
        base = wbase + ci * CE_D2
        pltpu.sync_copy(src_hbm.at[pl.ds(base, CE_D2)], src_v)
        pltpu.sync_copy(dst_hbm.at[pl.ds(base, CE_D2)], dst_v)

        @pl.loop(0, CE_D2 // 16)
        def _vec(j):
            sl = pl.ds(j * 16, 16)
            si = src_v[sl]
            di = dst_v[sl]
            dx = plsc.load_gather(px_v, [di]) - plsc.load_gather(px_v, [si])
            dy = plsc.load_gather(py_v, [di]) - plsc.load_gather(py_v, [si])
            dz = plsc.load_gather(pz_v, [di]) - plsc.load_gather(pz_v, [si])
            d2 = dx * dx + dy * dy + dz * dz
            eid = base + j * 16 + lanes
            d2_v[sl] = jnp.where(eid < E, d2, 100.0)

        pltpu.sync_copy(d2_v, d2_hbm.at[pl.ds(base, CE_D2)])


def _sc_d2(px, py, pz, src, dst):
    f = pl.kernel(
        _sc_d2_body,
        out_type=jax.ShapeDtypeStruct((E_PAD,), jnp.float32),
        mesh=_sc_mesh(),
        compiler_params=pltpu.CompilerParams(needs_layout_passes=False),
        scratch_types=[
            pltpu.VMEM((N_PADR,), jnp.float32),
            pltpu.VMEM((N_PADR,), jnp.float32),
            pltpu.VMEM((N_PADR,), jnp.float32),
            pltpu.VMEM((CE_D2,), jnp.int32),
            pltpu.VMEM((CE_D2,), jnp.int32),
            pltpu.VMEM((CE_D2,), jnp.float32),
        ],
    )
    return f(px, py, pz, src, dst)


def _sc_cfconv_body(h_hbm, wc_hbm, src_hbm, dst_hbm, out_hbm,
                    src_v, dst_v, hrows_v, wc_v, macc, sem):
    cid = lax.axis_index("c")
    sid = lax.axis_index("s")
    wid = sid * NC + cid

    @pl.loop(0, ZROWS)
    def _zr(i):
        for k2 in range(8):
            hrows_v[i, pl.ds(k2 * 16, 16)] = jnp.zeros((16,), jnp.float32)

    row0 = sid * ROWS_PER_TILE
    for k in range(ROWS_PER_TILE // ZROWS):
        pltpu.sync_copy(hrows_v.at[pl.ds(0, ZROWS)],
                        macc.at[pl.ds(row0 + k * ZROWS, ZROWS)])
    plsc.subcore_barrier()

    wbase = wid * EPW

    @pl.loop(0, EPW // CE)
    def _chunk(ci):
        base = wbase + ci * CE
        pltpu.sync_copy(src_hbm.at[pl.ds(base, CE)], src_v)
        pltpu.sync_copy(dst_hbm.at[pl.ds(base, CE)], dst_v)
        pltpu.async_copy(h_hbm.at[src_v], hrows_v, sem).wait()
        pltpu.sync_copy(wc_hbm.at[pl.ds(base, CE)], wc_v)

        @pl.loop(0, CE)
        def _row(i):
            for k2 in range(8):
                sl = pl.ds(k2 * 16, 16)
                hrows_v[i, sl] = hrows_v[i, sl] * wc_v[i, sl]

        pltpu.sync_copy(hrows_v, macc.at[dst_v], add=True)

    plsc.subcore_barrier()
    pltpu.sync_copy(macc.at[pl.ds(row0, ROWS_PER_TILE)],
                    out_hbm.at[cid, pl.ds(row0, ROWS_PER_TILE)])


def _sc_cfconv(h, wc, src, dst):
    f = pl.kernel(
        _sc_cfconv_body,
        out_type=jax.ShapeDtypeStruct((NC, N_PADR, NF), jnp.float32),
        mesh=_sc_mesh(),
        compiler_params=pltpu.CompilerParams(needs_layout_passes=False),
        scratch_types=[
            pltpu.VMEM((CE,), jnp.int32),
            pltpu.VMEM((CE,), jnp.int32),
            pltpu.VMEM((CE, NF), jnp.float32),
            pltpu.VMEM((CE, NF), jnp.float32),
            pltpu.VMEM_SHARED((N_PADR, NF), jnp.float32),
            pltpu.SemaphoreType.DMA,
        ],
    )
    return f(h, wc, src, dst)


def kernel(z, pos, edge_index, emb, Wf1, bf1, Wf2, bf2, Win, Wout1, bout1, Wout2, bout2, gamma, beta):
    src = jnp.pad(edge_index[0].astype(jnp.int32), (0, E_PAD - E))
    dst = jnp.pad(edge_index[1].astype(jnp.int32), (0, E_PAD - E))

    pos_t = jnp.pad(pos.T, ((0, 0), (0, N_PADR - N)))
    d2 = _sc_d2(pos_t[0], pos_t[1], pos_t[2], src, dst)
    d2_col = d2.reshape(E_PAD, 1)

    wf1p = jnp.pad(Wf1, ((0, 0), (0, NF - NB), (0, 0)))
    emb_pad = jnp.pad(emb, ((0, NF - emb.shape[0]), (0, 0)))
    b2 = lambda b: b.reshape(N_INTER, 1, NF)
    bf1c, bf2c, bout1c, bout2c = b2(bf1), b2(bf2), b2(bout1), b2(bout2)
    gammac, betac = b2(gamma), b2(beta)

    z_col = z.reshape(N, 1).astype(jnp.int32)
    feat, h = _init_feat(z_col, emb_pad, Win[0])

    for i in range(N_INTER):
        wc = _edge_filter(d2_col, wf1p[i], Wf2[i], bf1c[i], bf2c[i], E_PAD)
        m2 = _sc_cfconv(h, wc, src, dst)
        feat, h = _node_update(feat, m2[0, :N], m2[1, :N], Wout1[i], Wout2[i],
                               bout1c[i], bout2c[i], gammac[i], betac[i],
                               Win[(i + 1) % N_INTER])
    return feat
